# baseline (speedup 1.0000x reference)
import functools

import jax
import jax.numpy as jnp
from jax import lax
from jax.experimental import pallas as pl
from jax.experimental.pallas import tpu as pltpu
from jax.experimental.pallas import tpu_sc as plsc

NN = 100000
NE = 1600000
NG = 64
F = 32
NC, NS, L = 2, 16, 16
NW = NC * NS
NPT = NN // NW
LCAP = 1638400
CH_BIN = 1280
NCH_BIN = NE // CH_BIN
CH_G = 512
NCH_G = NE // CH_G
BSZ = 256
ACC_W = (NPT + 1) * F

_mesh = plsc.VectorSubcoreMesh(core_axis_name="c", subcore_axis_name="s")


def _wid():
    return lax.axis_index("s") * NC + lax.axis_index("c")


def _iota16():
    return lax.broadcasted_iota(jnp.int32, (L,), 0)


def _lane_bcast(v, e):
    idx = jnp.full((L, 1), e, jnp.int32)
    dnums = lax.GatherDimensionNumbers(
        offset_dims=(), collapsed_slice_dims=(0,), start_index_map=(0,))
    return lax.gather(v, idx, dnums, (1,),
                      mode=lax.GatherScatterMode.PROMISE_IN_BOUNDS)


def _bin_body(dst_hbm, ids_hbm, dls_hbm, cnt_hbm, dchunk, idbuf, dlbuf, cntb):
    wid = _wid()
    lo = wid * NPT
    hi = lo + NPT
    iota = _iota16()

    def _prefill(v, _):
        idbuf[pl.ds(v * L, L)] = v * L + iota
        dlbuf[pl.ds(v * L, L)] = jnp.full((L,), NPT, jnp.int32)
        return 0

    lax.fori_loop(0, CH_BIN // L, _prefill, 0)

    def _chunk(c, written):
        pltpu.sync_copy(dst_hbm.at[pl.ds(c * CH_BIN, CH_BIN)], dchunk)

        def _group(g, cnt_v):
            off = cnt_v
            for k in range(8):
                dv = dchunk[pl.ds(g * 128 + k * L, L)]
                m = (dv >= lo) & (dv < hi)
                ids = (c * CH_BIN + g * 128 + k * L) + iota
                pc = plsc.all_reduce_population_count(m)
                rank = plsc.cumsum(jnp.where(m, 1, 0)) - 1
                addr = off + rank
                plsc.store_scatter(idbuf, [addr], ids, mask=m)
                plsc.store_scatter(dlbuf, [addr], dv - lo, mask=m)
                off = off + pc
            return off

        cnt_v = lax.fori_loop(0, CH_BIN // 128, _group,
                              jnp.zeros((L,), jnp.int32))
        cnt = jnp.max(cnt_v)
        base = pl.multiple_of(wid * LCAP + written, 16)
        pltpu.sync_copy(idbuf, ids_hbm.at[pl.ds(base, CH_BIN)])
        pltpu.sync_copy(dlbuf, dls_hbm.at[pl.ds(base, CH_BIN)])
        return written + ((cnt + 15) & (-16))

    written = lax.fori_loop(0, NCH_BIN, _chunk, jnp.int32(0))
    base = pl.multiple_of(wid * LCAP + written, 16)
    pltpu.sync_copy(idbuf, ids_hbm.at[pl.ds(base, CH_BIN)])
    pltpu.sync_copy(dlbuf, dls_hbm.at[pl.ds(base, CH_BIN)])
    cntb[...] = jnp.full((L,), written, jnp.int32)
    pltpu.sync_copy(cntb, cnt_hbm.at[pl.ds(pl.multiple_of(wid * L, L), L)])


_bin_edges = pl.kernel(
    _bin_body,
    out_type=[
        jax.ShapeDtypeStruct((NW * LCAP,), jnp.int32),
        jax.ShapeDtypeStruct((NW * LCAP,), jnp.int32),
        jax.ShapeDtypeStruct((NW * L,), jnp.int32),
    ],
    mesh=_mesh,
    compiler_params=pltpu.CompilerParams(use_tc_tiling_on_sc=False, needs_layout_passes=False),
    scratch_types=[
        pltpu.VMEM((CH_BIN,), jnp.int32),
        pltpu.VMEM((CH_BIN,), jnp.int32),
        pltpu.VMEM((CH_BIN,), jnp.int32),
        pltpu.VMEM((L,), jnp.int32),
    ],
)


def _gather_body(src_hbm, dst_hbm, a_hbm, b_hbm, ag_hbm, bg_hbm,
                 sidx, didx, abuf, bbuf, sem):
    wid = _wid()
    nch = 97 + jnp.where(wid < NCH_G % NW, 1, 0)

    def _chunk(t, _):
        c = wid + NW * t
        ebase = pl.multiple_of(c * CH_G, CH_G)
        pltpu.sync_copy(src_hbm.at[pl.ds(ebase, CH_G)], sidx)
        pltpu.sync_copy(dst_hbm.at[pl.ds(ebase, CH_G)], didx)
        descs = []
        for k in range(CH_G // 128):
            s = pl.ds(k * 128, 128)
            descs.append(pltpu.async_copy(a_hbm.at[sidx.at[s]], abuf.at[s], sem))
            descs.append(pltpu.async_copy(b_hbm.at[didx.at[s]], bbuf.at[s], sem))
        for d in descs:
            d.wait()
        pltpu.sync_copy(abuf, ag_hbm.at[pl.ds(ebase, CH_G)])
        pltpu.sync_copy(bbuf, bg_hbm.at[pl.ds(ebase, CH_G)])
        return 0

    lax.fori_loop(0, nch, _chunk, 0)


_gather_rows = pl.kernel(
    _gather_body,
    out_type=[
        jax.ShapeDtypeStruct((NE, F), jnp.float32),
        jax.ShapeDtypeStruct((NE, F), jnp.float32),
    ],
    mesh=_mesh,
    compiler_params=pltpu.CompilerParams(use_tc_tiling_on_sc=False, needs_layout_passes=False),
    scratch_types=[
        pltpu.VMEM((CH_G,), jnp.int32),
        pltpu.VMEM((CH_G,), jnp.int32),
        pltpu.VMEM((CH_G, F), jnp.float32),
        pltpu.VMEM((CH_G, F), jnp.float32),
        pltpu.SemaphoreType.DMA,
    ],
)


def _scatter_body(m_hbm, ids_hbm, dls_hbm, cnt_hbm, h_hbm,
                  acc, mb, idb, dlb, cntb, sem):
    wid = _wid()
    iota = _iota16()

    def _zero(i, _):
        acc[pl.ds(i * L, L)] = jnp.zeros((L,), jnp.float32)
        return 0

    lax.fori_loop(0, ACC_W // L, _zero, 0)

    pltpu.sync_copy(cnt_hbm.at[pl.ds(pl.multiple_of(wid * L, L), L)], cntb)
    cnt = jnp.max(cntb[...])
    nb = (cnt + (BSZ - 1)) >> 8

    def _batch(b, _):
        base = pl.multiple_of(wid * LCAP + b * BSZ, BSZ)
        pltpu.sync_copy(ids_hbm.at[pl.ds(base, BSZ)], idb)
        pltpu.sync_copy(dls_hbm.at[pl.ds(base, BSZ)], dlb)
        d0 = pltpu.async_copy(m_hbm.at[idb.at[pl.ds(0, 128)]],
                              mb.at[pl.ds(0, 128)], sem)
        d1 = pltpu.async_copy(m_hbm.at[idb.at[pl.ds(128, 128)]],
                              mb.at[pl.ds(128, 128)], sem)
        d0.wait()
        d1.wait()

        def _sub(j, _):
            dlv = dlb[pl.ds(j * L, L)]
            for e in range(L):
                row = j * L + e
                bcr = jnp.full((L,), row, jnp.int32)
                m0 = plsc.load_gather(mb, [bcr, iota])
                m1 = plsc.load_gather(mb, [bcr, iota + L])
                a0 = _lane_bcast(dlv, e) * F + iota
                a1 = a0 + L
                old0 = plsc.load_gather(acc, [a0])
                plsc.store_scatter(acc, [a0], jnp.maximum(old0, m0))
                old1 = plsc.load_gather(acc, [a1])
                plsc.store_scatter(acc, [a1], jnp.maximum(old1, m1))
            return 0

        lax.fori_loop(0, BSZ // L, _sub, 0)
        return 0

    lax.fori_loop(0, nb, _batch, 0)
    pltpu.sync_copy(acc.at[pl.ds(0, NPT * F)],
                    h_hbm.at[pl.ds(pl.multiple_of(wid * NPT * F, 8), NPT * F)])


_scatter_max = pl.kernel(
    _scatter_body,
    out_type=jax.ShapeDtypeStruct((NN * F,), jnp.float32),
    mesh=_mesh,
    compiler_params=pltpu.CompilerParams(use_tc_tiling_on_sc=False, needs_layout_passes=False),
    scratch_types=[
        pltpu.VMEM((ACC_W,), jnp.float32),
        pltpu.VMEM((BSZ, F), jnp.float32),
        pltpu.VMEM((BSZ,), jnp.int32),
        pltpu.VMEM((BSZ,), jnp.int32),
        pltpu.VMEM((L,), jnp.int32),
        pltpu.SemaphoreType.DMA,
    ],
)


_POOL_CH = 512
_POOL_FULL = NN // _POOL_CH
_POOL_TAIL = NN - _POOL_FULL * _POOL_CH


def _pool_body(h_hbm, batch_hbm, parts_hbm, acc, bchunk, hb):
    wid = _wid()
    iota = _iota16()

    def _zero(i, _):
        acc[pl.ds(i * L, L)] = jnp.zeros((L,), jnp.float32)
        return 0

    lax.fori_loop(0, (NG * F) // L, _zero, 0)

    def _run(nodes, base):
        pltpu.sync_copy(batch_hbm.at[pl.ds(base, nodes)],
                        bchunk.at[pl.ds(0, nodes)])
        pltpu.sync_copy(h_hbm.at[pl.ds(base, nodes)], hb.at[pl.ds(0, nodes)])

        def _sub(j, _):
            bv = bchunk[pl.ds(j * L, L)]
            for e in range(L):
                row = j * L + e
                bcr = jnp.full((L,), row, jnp.int32)
                m0 = plsc.load_gather(hb, [bcr, iota])
                m1 = plsc.load_gather(hb, [bcr, iota + L])
                a0 = _lane_bcast(bv, e) * F + iota
                a1 = a0 + L
                old0 = plsc.load_gather(acc, [a0])
                plsc.store_scatter(acc, [a0], jnp.maximum(old0, m0))
                old1 = plsc.load_gather(acc, [a1])
                plsc.store_scatter(acc, [a1], jnp.maximum(old1, m1))
            return 0

        lax.fori_loop(0, nodes // L, _sub, 0)

    nch = 6 + jnp.where(wid < _POOL_FULL % NW, 1, 0)

    def _chunk(t, _):
        _run(_POOL_CH, pl.multiple_of((wid + NW * t) * _POOL_CH, _POOL_CH))
        return 0

    lax.fori_loop(0, nch, _chunk, 0)

    @pl.when(wid == 3)
    def _tail():
        _run(_POOL_TAIL, _POOL_FULL * _POOL_CH)

    pltpu.sync_copy(acc, parts_hbm.at[pl.ds(pl.multiple_of(wid * NG * F, 8), NG * F)])


_pool = pl.kernel(
    _pool_body,
    out_type=jax.ShapeDtypeStruct((NW * NG * F,), jnp.float32),
    mesh=_mesh,
    compiler_params=pltpu.CompilerParams(use_tc_tiling_on_sc=False, needs_layout_passes=False),
    scratch_types=[
        pltpu.VMEM((NG * F,), jnp.float32),
        pltpu.VMEM((_POOL_CH,), jnp.int32),
        pltpu.VMEM((_POOL_CH, F), jnp.float32),
    ],
)


def _prep_body(pos_ref, wsum_ref, b1a_ref, w36_ref, w2a3_ref,
               a1_ref, b1_ref, b2_ref):
    p = pos_ref[...]
    a1_ref[...] = jnp.dot(p, wsum_ref[...],
                          preferred_element_type=jnp.float32, precision=lax.Precision.HIGHEST) + b1a_ref[...]
    b1_ref[...] = jnp.dot(p, w36_ref[...], preferred_element_type=jnp.float32, precision=lax.Precision.HIGHEST)
    b2_ref[...] = jnp.dot(p, w2a3_ref[...], preferred_element_type=jnp.float32, precision=lax.Precision.HIGHEST)


def _prep(pos, wsum, b1a, w36, w2a3):
    bm = 4000
    grid = NN // bm
    return pl.pallas_call(
        _prep_body,
        grid=(grid,),
        in_specs=[
            pl.BlockSpec((bm, 3), lambda i: (i, 0)),
            pl.BlockSpec((3, F), lambda i: (0, 0)),
            pl.BlockSpec((1, F), lambda i: (0, 0)),
            pl.BlockSpec((3, F), lambda i: (0, 0)),
            pl.BlockSpec((3, F), lambda i: (0, 0)),
        ],
        out_specs=[
            pl.BlockSpec((bm, F), lambda i: (i, 0)),
            pl.BlockSpec((bm, F), lambda i: (i, 0)),
            pl.BlockSpec((bm, F), lambda i: (i, 0)),
        ],
        out_shape=[jax.ShapeDtypeStruct((NN, F), jnp.float32)] * 3,
    )(pos, wsum, b1a, w36, w2a3)


def _mlp_body(ag_ref, bg_ref, wb_ref, bb_ref, m_ref):
    z = ag_ref[...] - bg_ref[...]
    lz = jnp.where(z > 0, z, 0.01 * z)
    m_ref[...] = jnp.dot(lz, wb_ref[...],
                         preferred_element_type=jnp.float32, precision=lax.Precision.HIGHEST) + bb_ref[...]


def _edge_mlp(ag, bg, wb, bb):
    bm = 8000
    grid = NE // bm
    return pl.pallas_call(
        _mlp_body,
        grid=(grid,),
        in_specs=[
            pl.BlockSpec((bm, F), lambda i: (i, 0)),
            pl.BlockSpec((bm, F), lambda i: (i, 0)),
            pl.BlockSpec((F, F), lambda i: (0, 0)),
            pl.BlockSpec((1, F), lambda i: (0, 0)),
        ],
        out_specs=pl.BlockSpec((bm, F), lambda i: (i, 0)),
        out_shape=jax.ShapeDtypeStruct((NE, F), jnp.float32),
        compiler_params=pltpu.CompilerParams(
            dimension_semantics=("parallel",)),
    )(ag, bg, wb, bb)


def _a2_body(h_ref, w_ref, b2_ref, b2a_ref, a2_ref):
    a2_ref[...] = (jnp.dot(h_ref[...], w_ref[...],
                           preferred_element_type=jnp.float32, precision=lax.Precision.HIGHEST)
                   + b2_ref[...] + b2a_ref[...])


def _a2(h, w2a32, b2, b2a):
    bm = 4000
    grid = NN // bm
    return pl.pallas_call(
        _a2_body,
        grid=(grid,),
        in_specs=[
            pl.BlockSpec((bm, F), lambda i: (i, 0)),
            pl.BlockSpec((F, F), lambda i: (0, 0)),
            pl.BlockSpec((bm, F), lambda i: (i, 0)),
            pl.BlockSpec((1, F), lambda i: (0, 0)),
        ],
        out_specs=pl.BlockSpec((bm, F), lambda i: (i, 0)),
        out_shape=jax.ShapeDtypeStruct((NN, F), jnp.float32),
        compiler_params=pltpu.CompilerParams(
            dimension_semantics=("parallel",)),
    )(h, w2a32, b2, b2a)


def _final_body(parts_ref, wc_ref, bc_ref, o_ref):
    g = jnp.max(parts_ref[...], axis=0)
    o_ref[...] = jnp.dot(g, wc_ref[...],
                         preferred_element_type=jnp.float32, precision=lax.Precision.HIGHEST) + bc_ref[...]


def _final(parts, wc, bc):
    return pl.pallas_call(
        _final_body,
        out_shape=jax.ShapeDtypeStruct((NG, 3), jnp.float32),
    )(parts, wc, bc)


def kernel(pos, edge_index, batch, W1a, b1a, W1b, b1b, W2a, b2a, W2b, b2b,
           Wc, bc):
    ei = edge_index.astype(jnp.int32)
    src = ei[0]
    dst = ei[1]
    batch = batch.astype(jnp.int32)

    wsum = W1a[:3] + W1a[3:6]
    w36 = W1a[3:6]
    w2a32 = W2a[:F]
    w2a3 = W2a[F:F + 3]

    a1, b1, b2 = _prep(pos, wsum, b1a.reshape(1, F), w36, w2a3)
    ids, dls, cnts = _bin_edges(dst)

    ag1, bg1 = _gather_rows(src, dst, a1, b1)
    m1 = _edge_mlp(ag1, bg1, W1b, b1b.reshape(1, F))
    h = _scatter_max(m1, ids, dls, cnts).reshape(NN, F)

    a2 = _a2(h, w2a32, b2, b2a.reshape(1, F))
    ag2, bg2 = _gather_rows(src, dst, a2, b2)
    m2 = _edge_mlp(ag2, bg2, W2b, b2b.reshape(1, F))
    h2 = _scatter_max(m2, ids, dls, cnts).reshape(NN, F)

    parts = _pool(h2, batch).reshape(NW, NG, F)
    return _final(parts, Wc, bc.reshape(1, 3))


# speedup vs baseline: 1.0496x; 1.0496x over previous
import functools

import jax
import jax.numpy as jnp
from jax import lax
from jax.experimental import pallas as pl
from jax.experimental.pallas import tpu as pltpu
from jax.experimental.pallas import tpu_sc as plsc

NN = 100000
NE = 1600000
NG = 64
F = 32
NC, NS, L = 2, 16, 16
NW = NC * NS
NPT = NN // NW
LCAP = 1638400
CH_BIN = 1280
NCH_BIN = NE // CH_BIN
CH_G = 512
NCH_G = NE // CH_G
BSZ = 256
ACC_W = (NPT + 1) * F

_mesh = plsc.VectorSubcoreMesh(core_axis_name="c", subcore_axis_name="s")


def _wid():
    return lax.axis_index("s") * NC + lax.axis_index("c")


def _iota16():
    return lax.broadcasted_iota(jnp.int32, (L,), 0)


def _lane_bcast(v, e):
    idx = jnp.full((L, 1), e, jnp.int32)
    dnums = lax.GatherDimensionNumbers(
        offset_dims=(), collapsed_slice_dims=(0,), start_index_map=(0,))
    return lax.gather(v, idx, dnums, (1,),
                      mode=lax.GatherScatterMode.PROMISE_IN_BOUNDS)


def _bin_body(dst_hbm, ids_hbm, dls_hbm, cnt_hbm, dchunk, idbuf, dlbuf, cntb):
    wid = _wid()
    lo = wid * NPT
    hi = lo + NPT
    iota = _iota16()

    def _prefill(v, _):
        idbuf[pl.ds(v * L, L)] = v * L + iota
        dlbuf[pl.ds(v * L, L)] = jnp.full((L,), NPT, jnp.int32)
        return 0

    lax.fori_loop(0, CH_BIN // L, _prefill, 0)

    def _chunk(c, written):
        pltpu.sync_copy(dst_hbm.at[pl.ds(c * CH_BIN, CH_BIN)], dchunk)

        def _group(g, cnt_v):
            off = cnt_v
            for k in range(8):
                dv = dchunk[pl.ds(g * 128 + k * L, L)]
                m = (dv >= lo) & (dv < hi)
                ids = (c * CH_BIN + g * 128 + k * L) + iota
                pc = plsc.all_reduce_population_count(m)
                rank = plsc.cumsum(jnp.where(m, 1, 0)) - 1
                addr = off + rank
                plsc.store_scatter(idbuf, [addr], ids, mask=m)
                plsc.store_scatter(dlbuf, [addr], dv - lo, mask=m)
                off = off + pc
            return off

        cnt_v = lax.fori_loop(0, CH_BIN // 128, _group,
                              jnp.zeros((L,), jnp.int32))
        cnt = jnp.max(cnt_v)
        base = pl.multiple_of(wid * LCAP + written, 16)
        pltpu.sync_copy(idbuf, ids_hbm.at[pl.ds(base, CH_BIN)])
        pltpu.sync_copy(dlbuf, dls_hbm.at[pl.ds(base, CH_BIN)])
        return written + ((cnt + 15) & (-16))

    written = lax.fori_loop(0, NCH_BIN, _chunk, jnp.int32(0))
    base = pl.multiple_of(wid * LCAP + written, 16)
    pltpu.sync_copy(idbuf, ids_hbm.at[pl.ds(base, CH_BIN)])
    pltpu.sync_copy(dlbuf, dls_hbm.at[pl.ds(base, CH_BIN)])
    cntb[...] = jnp.full((L,), written, jnp.int32)
    pltpu.sync_copy(cntb, cnt_hbm.at[pl.ds(pl.multiple_of(wid * L, L), L)])


_bin_edges = pl.kernel(
    _bin_body,
    out_type=[
        jax.ShapeDtypeStruct((NW * LCAP,), jnp.int32),
        jax.ShapeDtypeStruct((NW * LCAP,), jnp.int32),
        jax.ShapeDtypeStruct((NW * L,), jnp.int32),
    ],
    mesh=_mesh,
    compiler_params=pltpu.CompilerParams(use_tc_tiling_on_sc=False, needs_layout_passes=False),
    scratch_types=[
        pltpu.VMEM((CH_BIN,), jnp.int32),
        pltpu.VMEM((CH_BIN,), jnp.int32),
        pltpu.VMEM((CH_BIN,), jnp.int32),
        pltpu.VMEM((L,), jnp.int32),
    ],
)


P8 = 8


def _gather3_body(src_hbm, dst_hbm, a_hbm, p_hbm, ag_hbm, pj_hbm, pi_hbm,
                  sidx, didx, abuf, pjbuf, pibuf, sem):
    wid = _wid()
    nch = 97 + jnp.where(wid < NCH_G % NW, 1, 0)

    def _chunk(t, _):
        c = wid + NW * t
        ebase = pl.multiple_of(c * CH_G, CH_G)
        pltpu.sync_copy(src_hbm.at[pl.ds(ebase, CH_G)], sidx)
        pltpu.sync_copy(dst_hbm.at[pl.ds(ebase, CH_G)], didx)
        descs = []
        for k in range(CH_G // 128):
            s = pl.ds(k * 128, 128)
            descs.append(pltpu.async_copy(a_hbm.at[sidx.at[s]], abuf.at[s], sem))
            descs.append(pltpu.async_copy(p_hbm.at[sidx.at[s]], pjbuf.at[s], sem))
            descs.append(pltpu.async_copy(p_hbm.at[didx.at[s]], pibuf.at[s], sem))
        for d in descs:
            d.wait()
        pltpu.sync_copy(abuf, ag_hbm.at[pl.ds(ebase, CH_G)])
        pltpu.sync_copy(pjbuf, pj_hbm.at[pl.ds(ebase, CH_G)])
        pltpu.sync_copy(pibuf, pi_hbm.at[pl.ds(ebase, CH_G)])
        return 0

    lax.fori_loop(0, nch, _chunk, 0)


_gather3 = pl.kernel(
    _gather3_body,
    out_type=[
        jax.ShapeDtypeStruct((NE, F), jnp.float32),
        jax.ShapeDtypeStruct((NE, P8), jnp.float32),
        jax.ShapeDtypeStruct((NE, P8), jnp.float32),
    ],
    mesh=_mesh,
    compiler_params=pltpu.CompilerParams(use_tc_tiling_on_sc=False, needs_layout_passes=False),
    scratch_types=[
        pltpu.VMEM((CH_G,), jnp.int32),
        pltpu.VMEM((CH_G,), jnp.int32),
        pltpu.VMEM((CH_G, F), jnp.float32),
        pltpu.VMEM((CH_G, P8), jnp.float32),
        pltpu.VMEM((CH_G, P8), jnp.float32),
        pltpu.SemaphoreType.DMA,
    ],
)


def _gather1_body(src_hbm, a_hbm, ag_hbm, sidx, abuf, sem):
    wid = _wid()
    nch = 97 + jnp.where(wid < NCH_G % NW, 1, 0)

    def _chunk(t, _):
        c = wid + NW * t
        ebase = pl.multiple_of(c * CH_G, CH_G)
        pltpu.sync_copy(src_hbm.at[pl.ds(ebase, CH_G)], sidx)
        descs = []
        for k in range(CH_G // 128):
            s = pl.ds(k * 128, 128)
            descs.append(pltpu.async_copy(a_hbm.at[sidx.at[s]], abuf.at[s], sem))
        for d in descs:
            d.wait()
        pltpu.sync_copy(abuf, ag_hbm.at[pl.ds(ebase, CH_G)])
        return 0

    lax.fori_loop(0, nch, _chunk, 0)


_gather1 = pl.kernel(
    _gather1_body,
    out_type=jax.ShapeDtypeStruct((NE, F), jnp.float32),
    mesh=_mesh,
    compiler_params=pltpu.CompilerParams(use_tc_tiling_on_sc=False, needs_layout_passes=False),
    scratch_types=[
        pltpu.VMEM((CH_G,), jnp.int32),
        pltpu.VMEM((CH_G, F), jnp.float32),
        pltpu.SemaphoreType.DMA,
    ],
)


def _scatter_body(m_hbm, ids_hbm, dls_hbm, cnt_hbm, h_hbm,
                  acc, mb, idb, dlb, cntb, sem):
    wid = _wid()
    iota = _iota16()

    def _zero(i, _):
        acc[pl.ds(i * L, L)] = jnp.zeros((L,), jnp.float32)
        return 0

    lax.fori_loop(0, ACC_W // L, _zero, 0)

    pltpu.sync_copy(cnt_hbm.at[pl.ds(pl.multiple_of(wid * L, L), L)], cntb)
    cnt = jnp.max(cntb[...])
    nb = (cnt + (BSZ - 1)) >> 8

    def _batch(b, _):
        base = pl.multiple_of(wid * LCAP + b * BSZ, BSZ)
        pltpu.sync_copy(ids_hbm.at[pl.ds(base, BSZ)], idb)
        pltpu.sync_copy(dls_hbm.at[pl.ds(base, BSZ)], dlb)
        d0 = pltpu.async_copy(m_hbm.at[idb.at[pl.ds(0, 128)]],
                              mb.at[pl.ds(0, 128)], sem)
        d1 = pltpu.async_copy(m_hbm.at[idb.at[pl.ds(128, 128)]],
                              mb.at[pl.ds(128, 128)], sem)
        d0.wait()
        d1.wait()

        def _sub(j, _):
            dlv = dlb[pl.ds(j * L, L)]
            for e in range(L):
                row = j * L + e
                m0 = mb[row, pl.ds(0, L)]
                m1 = mb[row, pl.ds(L, L)]
                a0 = dlv[e] * F + iota
                a1 = a0 + L
                old0 = plsc.load_gather(acc, [a0])
                plsc.store_scatter(acc, [a0], jnp.maximum(old0, m0))
                old1 = plsc.load_gather(acc, [a1])
                plsc.store_scatter(acc, [a1], jnp.maximum(old1, m1))
            return 0

        lax.fori_loop(0, BSZ // L, _sub, 0)
        return 0

    lax.fori_loop(0, nb, _batch, 0)
    pltpu.sync_copy(acc.at[pl.ds(0, NPT * F)],
                    h_hbm.at[pl.ds(pl.multiple_of(wid * NPT * F, 8), NPT * F)])


_scatter_max = pl.kernel(
    _scatter_body,
    out_type=jax.ShapeDtypeStruct((NN * F,), jnp.float32),
    mesh=_mesh,
    compiler_params=pltpu.CompilerParams(use_tc_tiling_on_sc=False, needs_layout_passes=False),
    scratch_types=[
        pltpu.VMEM((ACC_W,), jnp.float32),
        pltpu.VMEM((BSZ, F), jnp.float32),
        pltpu.VMEM((BSZ,), jnp.int32),
        pltpu.VMEM((BSZ,), jnp.int32),
        pltpu.VMEM((L,), jnp.int32),
        pltpu.SemaphoreType.DMA,
    ],
)


_POOL_CH = 512
_POOL_FULL = NN // _POOL_CH
_POOL_TAIL = NN - _POOL_FULL * _POOL_CH


def _pool_body(h_hbm, batch_hbm, parts_hbm, acc, bchunk, hb):
    wid = _wid()
    iota = _iota16()

    def _zero(i, _):
        acc[pl.ds(i * L, L)] = jnp.zeros((L,), jnp.float32)
        return 0

    lax.fori_loop(0, (NG * F) // L, _zero, 0)

    def _run(nodes, base):
        pltpu.sync_copy(batch_hbm.at[pl.ds(base, nodes)],
                        bchunk.at[pl.ds(0, nodes)])
        pltpu.sync_copy(h_hbm.at[pl.ds(base, nodes)], hb.at[pl.ds(0, nodes)])

        def _sub(j, _):
            bv = bchunk[pl.ds(j * L, L)]
            for e in range(L):
                row = j * L + e
                m0 = hb[row, pl.ds(0, L)]
                m1 = hb[row, pl.ds(L, L)]
                a0 = bv[e] * F + iota
                a1 = a0 + L
                old0 = plsc.load_gather(acc, [a0])
                plsc.store_scatter(acc, [a0], jnp.maximum(old0, m0))
                old1 = plsc.load_gather(acc, [a1])
                plsc.store_scatter(acc, [a1], jnp.maximum(old1, m1))
            return 0

        lax.fori_loop(0, nodes // L, _sub, 0)

    nch = 6 + jnp.where(wid < _POOL_FULL % NW, 1, 0)

    def _chunk(t, _):
        _run(_POOL_CH, pl.multiple_of((wid + NW * t) * _POOL_CH, _POOL_CH))
        return 0

    lax.fori_loop(0, nch, _chunk, 0)

    @pl.when(wid == 3)
    def _tail():
        _run(_POOL_TAIL, _POOL_FULL * _POOL_CH)

    pltpu.sync_copy(acc, parts_hbm.at[pl.ds(pl.multiple_of(wid * NG * F, 8), NG * F)])


_pool = pl.kernel(
    _pool_body,
    out_type=jax.ShapeDtypeStruct((NW * NG * F,), jnp.float32),
    mesh=_mesh,
    compiler_params=pltpu.CompilerParams(use_tc_tiling_on_sc=False, needs_layout_passes=False),
    scratch_types=[
        pltpu.VMEM((NG * F,), jnp.float32),
        pltpu.VMEM((_POOL_CH,), jnp.int32),
        pltpu.VMEM((_POOL_CH, F), jnp.float32),
    ],
)


def _bdot(x, w):
    return jnp.dot(x.astype(jnp.bfloat16), w.astype(jnp.bfloat16),
                   preferred_element_type=jnp.float32)


def _proj_body(x_ref, w_ref, b_ref, o_ref):
    o_ref[...] = _bdot(x_ref[...], w_ref[...]) + b_ref[...]


def _proj(x, w, b):
    bm = 4000
    k = x.shape[1]
    return pl.pallas_call(
        _proj_body,
        grid=(NN // bm,),
        in_specs=[
            pl.BlockSpec((bm, k), lambda i: (i, 0)),
            pl.BlockSpec((k, F), lambda i: (0, 0)),
            pl.BlockSpec((1, F), lambda i: (0, 0)),
        ],
        out_specs=pl.BlockSpec((bm, F), lambda i: (i, 0)),
        out_shape=jax.ShapeDtypeStruct((NN, F), jnp.float32),
        compiler_params=pltpu.CompilerParams(
            dimension_semantics=("parallel",)),
    )(x, w, b)


def _mlp_body(ag_ref, pj_ref, pi_ref, wp_ref, wb_ref, bb_ref, m_ref):
    d = pj_ref[...] - pi_ref[...]
    z = ag_ref[...] + _bdot(d, wp_ref[...])
    lz = jnp.where(z > 0, z, 0.01 * z)
    m_ref[...] = _bdot(lz, wb_ref[...]) + bb_ref[...]


def _edge_mlp(ag, pj, pi, wp8, wb, bb):
    bm = 8000
    return pl.pallas_call(
        _mlp_body,
        grid=(NE // bm,),
        in_specs=[
            pl.BlockSpec((bm, F), lambda i: (i, 0)),
            pl.BlockSpec((bm, P8), lambda i: (i, 0)),
            pl.BlockSpec((bm, P8), lambda i: (i, 0)),
            pl.BlockSpec((P8, F), lambda i: (0, 0)),
            pl.BlockSpec((F, F), lambda i: (0, 0)),
            pl.BlockSpec((1, F), lambda i: (0, 0)),
        ],
        out_specs=pl.BlockSpec((bm, F), lambda i: (i, 0)),
        out_shape=jax.ShapeDtypeStruct((NE, F), jnp.float32),
        compiler_params=pltpu.CompilerParams(
            dimension_semantics=("parallel",)),
    )(ag, pj, pi, wp8, wb, bb)


def _final_body(parts_ref, wc_ref, bc_ref, o_ref):
    g = jnp.max(parts_ref[...], axis=0)
    o_ref[...] = _bdot(g, wc_ref[...]) + bc_ref[...]


def _final(parts, wc, bc):
    return pl.pallas_call(
        _final_body,
        out_shape=jax.ShapeDtypeStruct((NG, 3), jnp.float32),
    )(parts, wc, bc)


def kernel(pos, edge_index, batch, W1a, b1a, W1b, b1b, W2a, b2a, W2b, b2b,
           Wc, bc):
    ei = edge_index.astype(jnp.int32)
    src = ei[0]
    dst = ei[1]
    batch = batch.astype(jnp.int32)

    pos8 = jnp.pad(pos, ((0, 0), (0, P8 - 3)))
    wp1 = jnp.pad(W1a[3:6], ((0, P8 - 3), (0, 0)))
    wp2 = jnp.pad(W2a[F:F + 3], ((0, P8 - 3), (0, 0)))

    a1 = _proj(pos, W1a[:3], b1a.reshape(1, F))
    ids, dls, cnts = _bin_edges(dst)

    ag1, pj, pi = _gather3(src, dst, a1, pos8)
    m1 = _edge_mlp(ag1, pj, pi, wp1, W1b, b1b.reshape(1, F))
    h = _scatter_max(m1, ids, dls, cnts).reshape(NN, F)

    a2 = _proj(h, W2a[:F], b2a.reshape(1, F))
    ag2 = _gather1(src, a2)
    m2 = _edge_mlp(ag2, pj, pi, wp2, W2b, b2b.reshape(1, F))
    h2 = _scatter_max(m2, ids, dls, cnts).reshape(NN, F)

    parts = _pool(h2, batch).reshape(NW, NG, F)
    return _final(parts, Wc, bc.reshape(1, 3))


# speedup vs baseline: 1.1025x; 1.0504x over previous
import functools

import jax
import jax.numpy as jnp
from jax import lax
from jax.experimental import pallas as pl
from jax.experimental.pallas import tpu as pltpu
from jax.experimental.pallas import tpu_sc as plsc

NN = 100000
NE = 1600000
NG = 64
F = 32
NC, NS, L = 2, 16, 16
NW = NC * NS
NPT = NN // NW
LCAP = 1638400
CH_BIN = 1280
NCH_BIN = NE // CH_BIN
CH_G = 640
NCH_G = NE // CH_G
BSZ = 512
BSZ_LOG = 9
ACC_W = (NPT + 1) * F

_mesh = plsc.VectorSubcoreMesh(core_axis_name="c", subcore_axis_name="s")


def _wid():
    return lax.axis_index("s") * NC + lax.axis_index("c")


def _iota16():
    return lax.broadcasted_iota(jnp.int32, (L,), 0)


def _lane_bcast(v, e):
    idx = jnp.full((L, 1), e, jnp.int32)
    dnums = lax.GatherDimensionNumbers(
        offset_dims=(), collapsed_slice_dims=(0,), start_index_map=(0,))
    return lax.gather(v, idx, dnums, (1,),
                      mode=lax.GatherScatterMode.PROMISE_IN_BOUNDS)


def _bin_body(dst_hbm, ids_hbm, dls_hbm, cnt_hbm, dchunk, idbuf, dlbuf, cntb):
    wid = _wid()
    lo = wid * NPT
    hi = lo + NPT
    iota = _iota16()

    def _prefill(v, _):
        idbuf[pl.ds(v * L, L)] = v * L + iota
        dlbuf[pl.ds(v * L, L)] = jnp.full((L,), NPT, jnp.int32)
        return 0

    lax.fori_loop(0, CH_BIN // L, _prefill, 0)

    def _chunk(c, written):
        pltpu.sync_copy(dst_hbm.at[pl.ds(c * CH_BIN, CH_BIN)], dchunk)

        def _group(g, cnt_v):
            off = cnt_v
            for k in range(8):
                dv = dchunk[pl.ds(g * 128 + k * L, L)]
                m = (dv >= lo) & (dv < hi)
                ids = (c * CH_BIN + g * 128 + k * L) + iota
                pc = plsc.all_reduce_population_count(m)
                rank = plsc.cumsum(jnp.where(m, 1, 0)) - 1
                addr = off + rank
                plsc.store_scatter(idbuf, [addr], ids, mask=m)
                plsc.store_scatter(dlbuf, [addr], dv - lo, mask=m)
                off = off + pc
            return off

        cnt_v = lax.fori_loop(0, CH_BIN // 128, _group,
                              jnp.zeros((L,), jnp.int32))
        cnt = jnp.max(cnt_v)
        base = pl.multiple_of(wid * LCAP + written, 16)
        pltpu.sync_copy(idbuf, ids_hbm.at[pl.ds(base, CH_BIN)])
        pltpu.sync_copy(dlbuf, dls_hbm.at[pl.ds(base, CH_BIN)])
        return written + ((cnt + 15) & (-16))

    written = lax.fori_loop(0, NCH_BIN, _chunk, jnp.int32(0))
    base = pl.multiple_of(wid * LCAP + written, 16)
    pltpu.sync_copy(idbuf, ids_hbm.at[pl.ds(base, CH_BIN)])
    pltpu.sync_copy(dlbuf, dls_hbm.at[pl.ds(base, CH_BIN)])
    cntb[...] = jnp.full((L,), written, jnp.int32)
    pltpu.sync_copy(cntb, cnt_hbm.at[pl.ds(pl.multiple_of(wid * L, L), L)])


_bin_edges = pl.kernel(
    _bin_body,
    out_type=[
        jax.ShapeDtypeStruct((NW * LCAP,), jnp.int32),
        jax.ShapeDtypeStruct((NW * LCAP,), jnp.int32),
        jax.ShapeDtypeStruct((NW * L,), jnp.int32),
    ],
    mesh=_mesh,
    compiler_params=pltpu.CompilerParams(use_tc_tiling_on_sc=False, needs_layout_passes=False),
    scratch_types=[
        pltpu.VMEM((CH_BIN,), jnp.int32),
        pltpu.VMEM((CH_BIN,), jnp.int32),
        pltpu.VMEM((CH_BIN,), jnp.int32),
        pltpu.VMEM((L,), jnp.int32),
    ],
)


P8 = 8


def _gather3_body(src_hbm, dst_hbm, a_hbm, p_hbm, ag_hbm, pj_hbm, pi_hbm,
                  sidx, didx, abuf, pjbuf, pibuf, sem):
    wid = _wid()
    nch = NCH_G // NW + jnp.where(wid < NCH_G % NW, 1, 0)

    def _chunk(t, _):
        c = wid + NW * t
        ebase = pl.multiple_of(c * CH_G, CH_G)
        pltpu.sync_copy(src_hbm.at[pl.ds(ebase, CH_G)], sidx)
        pltpu.sync_copy(dst_hbm.at[pl.ds(ebase, CH_G)], didx)
        descs = []
        for k in range(CH_G // 128):
            s = pl.ds(k * 128, 128)
            descs.append(pltpu.async_copy(a_hbm.at[sidx.at[s]], abuf.at[s], sem))
            descs.append(pltpu.async_copy(p_hbm.at[sidx.at[s]], pjbuf.at[s], sem))
            descs.append(pltpu.async_copy(p_hbm.at[didx.at[s]], pibuf.at[s], sem))
        for d in descs:
            d.wait()
        pltpu.sync_copy(abuf, ag_hbm.at[pl.ds(ebase, CH_G)])
        pltpu.sync_copy(pjbuf, pj_hbm.at[pl.ds(ebase, CH_G)])
        pltpu.sync_copy(pibuf, pi_hbm.at[pl.ds(ebase, CH_G)])
        return 0

    lax.fori_loop(0, nch, _chunk, 0)


_gather3 = pl.kernel(
    _gather3_body,
    out_type=[
        jax.ShapeDtypeStruct((NE, F), jnp.float32),
        jax.ShapeDtypeStruct((NE, P8), jnp.float32),
        jax.ShapeDtypeStruct((NE, P8), jnp.float32),
    ],
    mesh=_mesh,
    compiler_params=pltpu.CompilerParams(use_tc_tiling_on_sc=False, needs_layout_passes=False),
    scratch_types=[
        pltpu.VMEM((CH_G,), jnp.int32),
        pltpu.VMEM((CH_G,), jnp.int32),
        pltpu.VMEM((CH_G, F), jnp.float32),
        pltpu.VMEM((CH_G, P8), jnp.float32),
        pltpu.VMEM((CH_G, P8), jnp.float32),
        pltpu.SemaphoreType.DMA,
    ],
)


def _gather1_body(src_hbm, a_hbm, ag_hbm, sidx, abuf, sem):
    wid = _wid()
    nch = NCH_G // NW + jnp.where(wid < NCH_G % NW, 1, 0)

    def _chunk(t, _):
        c = wid + NW * t
        ebase = pl.multiple_of(c * CH_G, CH_G)
        pltpu.sync_copy(src_hbm.at[pl.ds(ebase, CH_G)], sidx)
        descs = []
        for k in range(CH_G // 128):
            s = pl.ds(k * 128, 128)
            descs.append(pltpu.async_copy(a_hbm.at[sidx.at[s]], abuf.at[s], sem))
        for d in descs:
            d.wait()
        pltpu.sync_copy(abuf, ag_hbm.at[pl.ds(ebase, CH_G)])
        return 0

    lax.fori_loop(0, nch, _chunk, 0)


_gather1 = pl.kernel(
    _gather1_body,
    out_type=jax.ShapeDtypeStruct((NE, F), jnp.float32),
    mesh=_mesh,
    compiler_params=pltpu.CompilerParams(use_tc_tiling_on_sc=False, needs_layout_passes=False),
    scratch_types=[
        pltpu.VMEM((CH_G,), jnp.int32),
        pltpu.VMEM((CH_G, F), jnp.float32),
        pltpu.SemaphoreType.DMA,
    ],
)


def _scatter_body(m_hbm, ids_hbm, dls_hbm, cnt_hbm, h_hbm,
                  acc, mb, idb, dlb, cntb, sem):
    wid = _wid()
    iota = _iota16()

    def _zero(i, _):
        acc[pl.ds(i * L, L)] = jnp.zeros((L,), jnp.float32)
        return 0

    lax.fori_loop(0, ACC_W // L, _zero, 0)

    pltpu.sync_copy(cnt_hbm.at[pl.ds(pl.multiple_of(wid * L, L), L)], cntb)
    cnt = jnp.max(cntb[...])
    nb = (cnt + (BSZ - 1)) >> BSZ_LOG

    def _batch(b, _):
        base = pl.multiple_of(wid * LCAP + b * BSZ, BSZ)
        pltpu.sync_copy(ids_hbm.at[pl.ds(base, BSZ)], idb)
        pltpu.sync_copy(dls_hbm.at[pl.ds(base, BSZ)], dlb)
        descs = []
        for k in range(BSZ // 128):
            s = pl.ds(k * 128, 128)
            descs.append(pltpu.async_copy(m_hbm.at[idb.at[s]], mb.at[s], sem))
        for d in descs:
            d.wait()

        def _sub(j, _):
            dlv = dlb[pl.ds(j * L, L)]
            for e in range(L):
                row = j * L + e
                m0 = mb[row, pl.ds(0, L)]
                m1 = mb[row, pl.ds(L, L)]
                a0 = dlv[e] * F + iota
                a1 = a0 + L
                old0 = plsc.load_gather(acc, [a0])
                plsc.store_scatter(acc, [a0], jnp.maximum(old0, m0))
                old1 = plsc.load_gather(acc, [a1])
                plsc.store_scatter(acc, [a1], jnp.maximum(old1, m1))
            return 0

        lax.fori_loop(0, BSZ // L, _sub, 0)
        return 0

    lax.fori_loop(0, nb, _batch, 0)
    pltpu.sync_copy(acc.at[pl.ds(0, NPT * F)],
                    h_hbm.at[pl.ds(pl.multiple_of(wid * NPT * F, 8), NPT * F)])


_scatter_max = pl.kernel(
    _scatter_body,
    out_type=jax.ShapeDtypeStruct((NN * F,), jnp.float32),
    mesh=_mesh,
    compiler_params=pltpu.CompilerParams(use_tc_tiling_on_sc=False, needs_layout_passes=False),
    scratch_types=[
        pltpu.VMEM((ACC_W,), jnp.float32),
        pltpu.VMEM((BSZ, F), jnp.float32),
        pltpu.VMEM((BSZ,), jnp.int32),
        pltpu.VMEM((BSZ,), jnp.int32),
        pltpu.VMEM((L,), jnp.int32),
        pltpu.SemaphoreType.DMA,
    ],
)


_POOL_CH = 512
_POOL_FULL = NN // _POOL_CH
_POOL_TAIL = NN - _POOL_FULL * _POOL_CH


def _pool_body(h_hbm, batch_hbm, parts_hbm, acc, bchunk, hb):
    wid = _wid()
    iota = _iota16()

    def _zero(i, _):
        acc[pl.ds(i * L, L)] = jnp.zeros((L,), jnp.float32)
        return 0

    lax.fori_loop(0, (NG * F) // L, _zero, 0)

    def _run(nodes, base):
        pltpu.sync_copy(batch_hbm.at[pl.ds(base, nodes)],
                        bchunk.at[pl.ds(0, nodes)])
        pltpu.sync_copy(h_hbm.at[pl.ds(base, nodes)], hb.at[pl.ds(0, nodes)])

        def _sub(j, _):
            bv = bchunk[pl.ds(j * L, L)]
            for e in range(L):
                row = j * L + e
                m0 = hb[row, pl.ds(0, L)]
                m1 = hb[row, pl.ds(L, L)]
                a0 = bv[e] * F + iota
                a1 = a0 + L
                old0 = plsc.load_gather(acc, [a0])
                plsc.store_scatter(acc, [a0], jnp.maximum(old0, m0))
                old1 = plsc.load_gather(acc, [a1])
                plsc.store_scatter(acc, [a1], jnp.maximum(old1, m1))
            return 0

        lax.fori_loop(0, nodes // L, _sub, 0)

    nch = 6 + jnp.where(wid < _POOL_FULL % NW, 1, 0)

    def _chunk(t, _):
        _run(_POOL_CH, pl.multiple_of((wid + NW * t) * _POOL_CH, _POOL_CH))
        return 0

    lax.fori_loop(0, nch, _chunk, 0)

    @pl.when(wid == 3)
    def _tail():
        _run(_POOL_TAIL, _POOL_FULL * _POOL_CH)

    pltpu.sync_copy(acc, parts_hbm.at[pl.ds(pl.multiple_of(wid * NG * F, 8), NG * F)])


_pool = pl.kernel(
    _pool_body,
    out_type=jax.ShapeDtypeStruct((NW * NG * F,), jnp.float32),
    mesh=_mesh,
    compiler_params=pltpu.CompilerParams(use_tc_tiling_on_sc=False, needs_layout_passes=False),
    scratch_types=[
        pltpu.VMEM((NG * F,), jnp.float32),
        pltpu.VMEM((_POOL_CH,), jnp.int32),
        pltpu.VMEM((_POOL_CH, F), jnp.float32),
    ],
)


def _bdot(x, w):
    return jnp.dot(x.astype(jnp.bfloat16), w.astype(jnp.bfloat16),
                   preferred_element_type=jnp.float32)


def _proj_body(x_ref, w_ref, b_ref, o_ref):
    o_ref[...] = _bdot(x_ref[...], w_ref[...]) + b_ref[...]


def _proj(x, w, b):
    bm = 4000
    k = x.shape[1]
    return pl.pallas_call(
        _proj_body,
        grid=(NN // bm,),
        in_specs=[
            pl.BlockSpec((bm, k), lambda i: (i, 0)),
            pl.BlockSpec((k, F), lambda i: (0, 0)),
            pl.BlockSpec((1, F), lambda i: (0, 0)),
        ],
        out_specs=pl.BlockSpec((bm, F), lambda i: (i, 0)),
        out_shape=jax.ShapeDtypeStruct((NN, F), jnp.float32),
        compiler_params=pltpu.CompilerParams(
            dimension_semantics=("parallel",)),
    )(x, w, b)


def _mlp_body(ag_ref, pj_ref, pi_ref, wp_ref, wb_ref, bb_ref, m_ref):
    d = pj_ref[...] - pi_ref[...]
    z = ag_ref[...] + _bdot(d, wp_ref[...])
    lz = jnp.where(z > 0, z, 0.01 * z)
    m_ref[...] = _bdot(lz, wb_ref[...]) + bb_ref[...]


def _edge_mlp(ag, pj, pi, wp8, wb, bb):
    bm = 8000
    return pl.pallas_call(
        _mlp_body,
        grid=(NE // bm,),
        in_specs=[
            pl.BlockSpec((bm, F), lambda i: (i, 0)),
            pl.BlockSpec((bm, P8), lambda i: (i, 0)),
            pl.BlockSpec((bm, P8), lambda i: (i, 0)),
            pl.BlockSpec((P8, F), lambda i: (0, 0)),
            pl.BlockSpec((F, F), lambda i: (0, 0)),
            pl.BlockSpec((1, F), lambda i: (0, 0)),
        ],
        out_specs=pl.BlockSpec((bm, F), lambda i: (i, 0)),
        out_shape=jax.ShapeDtypeStruct((NE, F), jnp.float32),
        compiler_params=pltpu.CompilerParams(
            dimension_semantics=("parallel",)),
    )(ag, pj, pi, wp8, wb, bb)


def _final_body(parts_ref, wc_ref, bc_ref, o_ref):
    g = jnp.max(parts_ref[...], axis=0)
    o_ref[...] = _bdot(g, wc_ref[...]) + bc_ref[...]


def _final(parts, wc, bc):
    return pl.pallas_call(
        _final_body,
        out_shape=jax.ShapeDtypeStruct((NG, 3), jnp.float32),
    )(parts, wc, bc)


def kernel(pos, edge_index, batch, W1a, b1a, W1b, b1b, W2a, b2a, W2b, b2b,
           Wc, bc):
    ei = edge_index.astype(jnp.int32)
    src = ei[0]
    dst = ei[1]
    batch = batch.astype(jnp.int32)

    pos8 = jnp.pad(pos, ((0, 0), (0, P8 - 3)))
    wp1 = jnp.pad(W1a[3:6], ((0, P8 - 3), (0, 0)))
    wp2 = jnp.pad(W2a[F:F + 3], ((0, P8 - 3), (0, 0)))

    a1 = _proj(pos, W1a[:3], b1a.reshape(1, F))
    ids, dls, cnts = _bin_edges(dst)

    ag1, pj, pi = _gather3(src, dst, a1, pos8)
    m1 = _edge_mlp(ag1, pj, pi, wp1, W1b, b1b.reshape(1, F))
    h = _scatter_max(m1, ids, dls, cnts).reshape(NN, F)

    a2 = _proj(h, W2a[:F], b2a.reshape(1, F))
    ag2 = _gather1(src, a2)
    m2 = _edge_mlp(ag2, pj, pi, wp2, W2b, b2b.reshape(1, F))
    h2 = _scatter_max(m2, ids, dls, cnts).reshape(NN, F)

    parts = _pool(h2, batch).reshape(NW, NG, F)
    return _final(parts, Wc, bc.reshape(1, 3))
